# Initial kernel scaffold; baseline (speedup 1.0000x reference)
import functools

import jax
import jax.numpy as jnp
from jax.experimental import pallas as pl
from jax.experimental.pallas import tpu as pltpu

N = 1024
KNN = 8
NEG = -3.0e38


def _edge_layer(xt, W, g1, g2, ci, co):
    f32 = jnp.float32
    xx = jnp.sum(xt * xt, axis=1, keepdims=True)
    inner = -2.0 * jax.lax.dot_general(
        xt, xt, (((1,), (1,)), ((), ())), preferred_element_type=f32)
    pd = (-xx.reshape(1, N)) - inner - xx

    Wa = W[:, :ci]
    Wd = W[:, ci:] - Wa
    u = jax.lax.dot_general(xt, Wa, (((1,), (1,)), ((), ())),
                            preferred_element_type=f32)
    v = jax.lax.dot_general(xt, Wd, (((1,), (1,)), ((), ())),
                            preferred_element_type=f32)

    col = jax.lax.broadcasted_iota(jnp.int32, (N, N), 1)
    work = pd
    smax = jnp.full((N, co), NEG, f32)
    smin = jnp.full((N, co), -NEG, f32)
    s1 = jnp.zeros((N, co), f32)
    s2 = jnp.zeros((N, co), f32)
    for _ in range(KNN):
        m = jnp.max(work, axis=1, keepdims=True)
        eq = work == m
        jstar = jnp.min(jnp.where(eq, col, N), axis=1, keepdims=True)
        H = col == jstar
        g = jax.lax.dot_general(H.astype(f32), u, (((1,), (0,)), ((), ())),
                                preferred_element_type=f32)
        smax = jnp.maximum(smax, g)
        smin = jnp.minimum(smin, g)
        s1 = s1 + g
        s2 = s2 + g * g
        work = jnp.where(H, NEG, work)

    cnt = float(N * KNN)
    sum_v = jnp.sum(v, axis=0, keepdims=True)
    sum_z = jnp.sum(s1, axis=0, keepdims=True) + KNN * sum_v
    sum_z2 = (jnp.sum(s2, axis=0, keepdims=True)
              + 2.0 * jnp.sum(v * s1, axis=0, keepdims=True)
              + KNN * jnp.sum(v * v, axis=0, keepdims=True))
    mean = sum_z / cnt
    var = sum_z2 / cnt - mean * mean
    rstd = jax.lax.rsqrt(var + 1e-5)
    a = g1 * rstd
    b = g2 - a * mean
    zsel = v + jnp.where(a >= 0.0, smax, smin)
    y = a * zsel + b
    y = jnp.where(y >= 0.0, y, 0.2 * y)
    o = jnp.max(y, axis=0, keepdims=True)
    return y, o


def _fwd_kernel(x_ref, g11, g12, g21, g22, g31, g32, g41, g42,
                w1_ref, w2_ref, w3_ref, w4_ref, out_ref):
    xt = x_ref[0].T
    y1, o1 = _edge_layer(xt, w1_ref[...], g11[...], g12[...], 3, 64)
    y2, o2 = _edge_layer(y1, w2_ref[...], g21[...], g22[...], 64, 128)
    y3, o3 = _edge_layer(y2, w3_ref[...], g31[...], g32[...], 128, 256)
    _, o4 = _edge_layer(y3, w4_ref[...], g41[...], g42[...], 256, 512)
    out_ref[...] = jnp.concatenate([o1, o2, o3, o4], axis=1)


def kernel(x, style, W1, S1w, S1b, F1a, F1b, W2, S2w, S2b, F2a, F2b,
           W3, S3w, S3b, F3a, F3b, W4, S4w, S4b, F4a, F4b):
    B = x.shape[0]
    gs = []
    for (Sw, Sb, Fa, Fb) in ((S1w, S1b, F1a, F1b), (S2w, S2b, F2a, F2b),
                             (S3w, S3b, F3a, F3b), (S4w, S4b, F4a, F4b)):
        co = Sw.shape[0] // 2
        st = style @ Sw.T + Sb
        gamma, beta = st[:, :co], st[:, co:]
        s = jax.nn.sigmoid(jnp.maximum(beta @ Fa.T, 0.0) @ Fb.T)
        gs.append(s * gamma)
        gs.append(s * beta)

    full = lambda shp: pl.BlockSpec(shp, lambda b: (0, 0))
    row = lambda c: pl.BlockSpec((1, c), lambda b: (b, 0))
    grid_spec = pl.GridSpec(
        grid=(B,),
        in_specs=[pl.BlockSpec((1, 3, N), lambda b: (b, 0, 0)),
                  row(64), row(64), row(128), row(128),
                  row(256), row(256), row(512), row(512),
                  full((64, 6)), full((128, 128)),
                  full((256, 256)), full((512, 512))],
        out_specs=pl.BlockSpec((1, 960), lambda b: (b, 0)),
    )
    out = pl.pallas_call(
        _fwd_kernel,
        grid_spec=grid_spec,
        out_shape=jax.ShapeDtypeStruct((B, 960), jnp.float32),
        compiler_params=pltpu.CompilerParams(
            dimension_semantics=("arbitrary",)),
    )(x, *gs, W1, W2, W3, W4)
    return out


# speedup vs baseline: 1.8652x; 1.8652x over previous
import functools

import jax
import jax.numpy as jnp
from jax.experimental import pallas as pl
from jax.experimental.pallas import tpu as pltpu

N = 1024
KNN = 8
PB = 256
NB = N // PB
NEG = -3.0e38
POS = 3.0e38


def _topk_kernel(xt_ref, xxr_ref, idx_ref, pd_ref, ia_ref, *, ci):
    f32 = jnp.float32
    p = pl.program_id(1)
    r = pl.program_id(2)

    @pl.when(r == 0)
    def _():
        xt = xt_ref[0]
        xtb = xt_ref[0, pl.ds(p * PB, PB), :]
        xxb = jnp.sum(xtb * xtb, axis=1, keepdims=True)
        inner = -2.0 * jax.lax.dot_general(
            xtb, xt, (((1,), (1,)), ((), ())), preferred_element_type=f32)
        pd_ref[...] = (-xxr_ref[0]) - inner - xxb
        ia_ref[...] = jnp.zeros((PB, KNN), f32)

    w = pd_ref[...]
    m = jnp.max(w, axis=1, keepdims=True)
    col = jax.lax.broadcasted_iota(jnp.int32, (PB, N), 1)
    jstar = jnp.min(jnp.where(w == m, col, N), axis=1, keepdims=True)
    H = col == jstar
    pd_ref[...] = jnp.where(H, NEG, w)
    col8 = jax.lax.broadcasted_iota(jnp.int32, (PB, KNN), 1)
    ia_ref[...] = ia_ref[...] + jnp.where(col8 == r, jstar.astype(f32), 0.0)

    @pl.when(r == KNN - 1)
    def _():
        idx_ref[0] = ia_ref[...]


def _conv_kernel(xt_ref, ft_ref, w_ref, zmax_ref, zmin_ref, s1_ref, s2_ref,
                 *, ci, co):
    f32 = jnp.float32
    p = pl.program_id(1)
    r = pl.program_id(2)

    @pl.when(jnp.logical_and(p == 0, r == 0))
    def _():
        s1_ref[...] = jnp.zeros((1, 1, co), f32)
        s2_ref[...] = jnp.zeros((1, 1, co), f32)

    @pl.when(r == 0)
    def _():
        zmax_ref[...] = jnp.full((1, PB, co), NEG, f32)
        zmin_ref[...] = jnp.full((1, PB, co), POS, f32)

    feat = ft_ref[0, 0]
    xtb = xt_ref[0, pl.ds(p * PB, PB), :]
    edge = jnp.concatenate([feat - xtb, xtb], axis=1)
    z = jax.lax.dot_general(edge, w_ref[...], (((1,), (1,)), ((), ())),
                            preferred_element_type=f32)
    zmax_ref[0] = jnp.maximum(zmax_ref[0], z)
    zmin_ref[0] = jnp.minimum(zmin_ref[0], z)
    s1_ref[0] = s1_ref[0] + jnp.sum(z, axis=0, keepdims=True)
    s2_ref[0] = s2_ref[0] + jnp.sum(z * z, axis=0, keepdims=True)


def _finish_kernel(zmax_ref, zmin_ref, s1_ref, s2_ref, gam_ref, bet_ref,
                   se_ref, y_ref, o_ref, *, co):
    cnt = float(N * KNN)
    gam = gam_ref[0]
    bet = bet_ref[0]
    se = se_ref[0]
    mean = s1_ref[0] / cnt
    var = s2_ref[0] / cnt - mean * mean
    denom = jnp.sqrt(var + 1e-5)
    zsel = jnp.where(gam >= 0.0, zmax_ref[0], zmin_ref[0])
    norm = (zsel - mean) / denom
    y = (gam * norm + bet) * se
    y = jnp.where(y >= 0.0, y, 0.2 * y)
    y_ref[0] = y
    o_ref[0] = jnp.max(y, axis=0, keepdims=True)


def _layer_call(xt, gam, bet, se, W, ci, co):
    B = xt.shape[0]
    f32 = jnp.float32
    xxr = jnp.sum(xt * xt, axis=2)[:, None, :]
    idxf = pl.pallas_call(
        functools.partial(_topk_kernel, ci=ci),
        grid=(B, NB, KNN),
        in_specs=[pl.BlockSpec((1, N, ci), lambda b, p, r: (b, 0, 0)),
                  pl.BlockSpec((1, 1, N), lambda b, p, r: (b, 0, 0))],
        out_specs=pl.BlockSpec((1, PB, KNN), lambda b, p, r: (b, p, 0)),
        out_shape=jax.ShapeDtypeStruct((B, N, KNN), f32),
        scratch_shapes=[pltpu.VMEM((PB, N), f32),
                        pltpu.VMEM((PB, KNN), f32)],
        compiler_params=pltpu.CompilerParams(
            dimension_semantics=("arbitrary", "arbitrary", "arbitrary")),
    )(xt, xxr)
    idx = jnp.transpose(idxf.astype(jnp.int32), (0, 2, 1))
    feat = xt[jnp.arange(B)[:, None, None], idx]
    stats = pl.pallas_call(
        functools.partial(_conv_kernel, ci=ci, co=co),
        grid=(B, NB, KNN),
        in_specs=[pl.BlockSpec((1, N, ci), lambda b, p, r: (b, 0, 0)),
                  pl.BlockSpec((1, 1, PB, ci), lambda b, p, r: (b, r, p, 0)),
                  pl.BlockSpec((co, 2 * ci), lambda b, p, r: (0, 0))],
        out_specs=[pl.BlockSpec((1, PB, co), lambda b, p, r: (b, p, 0)),
                   pl.BlockSpec((1, PB, co), lambda b, p, r: (b, p, 0)),
                   pl.BlockSpec((1, 1, co), lambda b, p, r: (b, 0, 0)),
                   pl.BlockSpec((1, 1, co), lambda b, p, r: (b, 0, 0))],
        out_shape=[jax.ShapeDtypeStruct((B, N, co), f32),
                   jax.ShapeDtypeStruct((B, N, co), f32),
                   jax.ShapeDtypeStruct((B, 1, co), f32),
                   jax.ShapeDtypeStruct((B, 1, co), f32)],
        compiler_params=pltpu.CompilerParams(
            dimension_semantics=("arbitrary", "arbitrary", "arbitrary")),
    )(xt, feat, W)
    zmax, zmin, s1, s2 = stats
    row1 = lambda: pl.BlockSpec((1, 1, co), lambda b: (b, 0, 0))
    y, o = pl.pallas_call(
        functools.partial(_finish_kernel, co=co),
        grid=(B,),
        in_specs=[pl.BlockSpec((1, N, co), lambda b: (b, 0, 0)),
                  pl.BlockSpec((1, N, co), lambda b: (b, 0, 0)),
                  row1(), row1(), row1(), row1(), row1()],
        out_specs=[pl.BlockSpec((1, N, co), lambda b: (b, 0, 0)),
                   row1()],
        out_shape=[jax.ShapeDtypeStruct((B, N, co), f32),
                   jax.ShapeDtypeStruct((B, 1, co), f32)],
        compiler_params=pltpu.CompilerParams(
            dimension_semantics=("arbitrary",)),
    )(zmax, zmin, s1, s2, gam, bet, se)
    return y, o


def kernel(x, style, W1, S1w, S1b, F1a, F1b, W2, S2w, S2b, F2a, F2b,
           W3, S3w, S3b, F3a, F3b, W4, S4w, S4b, F4a, F4b):
    B = x.shape[0]
    gs = []
    for (Sw, Sb, Fa, Fb) in ((S1w, S1b, F1a, F1b), (S2w, S2b, F2a, F2b),
                             (S3w, S3b, F3a, F3b), (S4w, S4b, F4a, F4b)):
        co = Sw.shape[0] // 2
        st = style @ Sw.T + Sb
        gamma, beta = st[:, :co], st[:, co:]
        s = jax.nn.sigmoid(jnp.maximum(beta @ Fa.T, 0.0) @ Fb.T)
        gs.append((gamma.reshape(B, 1, co), beta.reshape(B, 1, co),
                   s.reshape(B, 1, co)))

    xt = jnp.transpose(x, (0, 2, 1))
    y1, o1 = _layer_call(xt, *gs[0], W1, 3, 64)
    y2, o2 = _layer_call(y1, *gs[1], W2, 64, 128)
    y3, o3 = _layer_call(y2, *gs[2], W3, 128, 256)
    _, o4 = _layer_call(y3, *gs[3], W4, 256, 512)
    return jnp.concatenate([o1[:, 0], o2[:, 0], o3[:, 0], o4[:, 0]], axis=1)


# speedup vs baseline: 2.0430x; 1.0953x over previous
import functools

import jax
import jax.numpy as jnp
from jax.experimental import pallas as pl
from jax.experimental.pallas import tpu as pltpu

N = 1024
KNN = 8
PB = 512
NB = N // PB
NEG = -3.0e38
POS = 3.0e38


def _topk_kernel(xt_ref, xxr_ref, idx_ref, pd_ref, ia_ref, *, ci):
    f32 = jnp.float32
    p = pl.program_id(1)
    r = pl.program_id(2)

    @pl.when(r == 0)
    def _():
        xt = xt_ref[0]
        xtb = xt_ref[0, pl.ds(p * PB, PB), :]
        xxb = jnp.sum(xtb * xtb, axis=1, keepdims=True)
        inner = -2.0 * jax.lax.dot_general(
            xtb, xt, (((1,), (1,)), ((), ())), preferred_element_type=f32)
        pd_ref[...] = (-xxr_ref[0]) - inner - xxb
        ia_ref[...] = jnp.zeros((PB, KNN), f32)

    w = pd_ref[...]
    m = jnp.max(w, axis=1, keepdims=True)
    col = jax.lax.broadcasted_iota(jnp.int32, (PB, N), 1)
    jstar = jnp.min(jnp.where(w == m, col, N), axis=1, keepdims=True)
    H = col == jstar
    pd_ref[...] = jnp.where(H, NEG, w)
    col8 = jax.lax.broadcasted_iota(jnp.int32, (PB, KNN), 1)
    ia_ref[...] = ia_ref[...] + jnp.where(col8 == r, jstar.astype(f32), 0.0)

    @pl.when(r == KNN - 1)
    def _():
        idx_ref[0] = ia_ref[...]


def _conv_kernel(xt_ref, ft_ref, w_ref, zmax_ref, zmin_ref, s1_ref, s2_ref,
                 *, ci, co):
    f32 = jnp.float32
    p = pl.program_id(1)
    r = pl.program_id(2)

    @pl.when(jnp.logical_and(p == 0, r == 0))
    def _():
        s1_ref[...] = jnp.zeros((1, 1, co), f32)
        s2_ref[...] = jnp.zeros((1, 1, co), f32)

    @pl.when(r == 0)
    def _():
        zmax_ref[...] = jnp.full((1, PB, co), NEG, f32)
        zmin_ref[...] = jnp.full((1, PB, co), POS, f32)

    feat = ft_ref[0, 0]
    xtb = xt_ref[0, pl.ds(p * PB, PB), :]
    edge = jnp.concatenate([feat - xtb, xtb], axis=1)
    z = jax.lax.dot_general(edge, w_ref[...], (((1,), (1,)), ((), ())),
                            preferred_element_type=f32)
    zmax_ref[0] = jnp.maximum(zmax_ref[0], z)
    zmin_ref[0] = jnp.minimum(zmin_ref[0], z)
    s1_ref[0] = s1_ref[0] + jnp.sum(z, axis=0, keepdims=True)
    s2_ref[0] = s2_ref[0] + jnp.sum(z * z, axis=0, keepdims=True)


def _finish_kernel(zmax_ref, zmin_ref, s1_ref, s2_ref, gam_ref, bet_ref,
                   se_ref, y_ref, o_ref, *, co):
    cnt = float(N * KNN)
    gam = gam_ref[0]
    bet = bet_ref[0]
    se = se_ref[0]
    mean = s1_ref[0] / cnt
    var = s2_ref[0] / cnt - mean * mean
    denom = jnp.sqrt(var + 1e-5)
    zsel = jnp.where(gam >= 0.0, zmax_ref[0], zmin_ref[0])
    norm = (zsel - mean) / denom
    y = (gam * norm + bet) * se
    y = jnp.where(y >= 0.0, y, 0.2 * y)
    y_ref[0] = y
    o_ref[0] = jnp.max(y, axis=0, keepdims=True)


def _layer_call(xt, gam, bet, se, W, ci, co):
    B = xt.shape[0]
    f32 = jnp.float32
    xxr = jnp.sum(xt * xt, axis=2)[:, None, :]
    idxf = pl.pallas_call(
        functools.partial(_topk_kernel, ci=ci),
        grid=(B, NB, KNN),
        in_specs=[pl.BlockSpec((1, N, ci), lambda b, p, r: (b, 0, 0)),
                  pl.BlockSpec((1, 1, N), lambda b, p, r: (b, 0, 0))],
        out_specs=pl.BlockSpec((1, PB, KNN), lambda b, p, r: (b, p, 0)),
        out_shape=jax.ShapeDtypeStruct((B, N, KNN), f32),
        scratch_shapes=[pltpu.VMEM((PB, N), f32),
                        pltpu.VMEM((PB, KNN), f32)],
        compiler_params=pltpu.CompilerParams(
            dimension_semantics=("arbitrary", "arbitrary", "arbitrary")),
    )(xt, xxr)
    idx = jnp.transpose(idxf.astype(jnp.int32), (0, 2, 1))
    feat = xt[jnp.arange(B)[:, None, None], idx]
    stats = pl.pallas_call(
        functools.partial(_conv_kernel, ci=ci, co=co),
        grid=(B, NB, KNN),
        in_specs=[pl.BlockSpec((1, N, ci), lambda b, p, r: (b, 0, 0)),
                  pl.BlockSpec((1, 1, PB, ci), lambda b, p, r: (b, r, p, 0)),
                  pl.BlockSpec((co, 2 * ci), lambda b, p, r: (0, 0))],
        out_specs=[pl.BlockSpec((1, PB, co), lambda b, p, r: (b, p, 0)),
                   pl.BlockSpec((1, PB, co), lambda b, p, r: (b, p, 0)),
                   pl.BlockSpec((1, 1, co), lambda b, p, r: (b, 0, 0)),
                   pl.BlockSpec((1, 1, co), lambda b, p, r: (b, 0, 0))],
        out_shape=[jax.ShapeDtypeStruct((B, N, co), f32),
                   jax.ShapeDtypeStruct((B, N, co), f32),
                   jax.ShapeDtypeStruct((B, 1, co), f32),
                   jax.ShapeDtypeStruct((B, 1, co), f32)],
        compiler_params=pltpu.CompilerParams(
            dimension_semantics=("arbitrary", "arbitrary", "arbitrary")),
    )(xt, feat, W)
    zmax, zmin, s1, s2 = stats
    row1 = lambda: pl.BlockSpec((1, 1, co), lambda b: (b, 0, 0))
    y, o = pl.pallas_call(
        functools.partial(_finish_kernel, co=co),
        grid=(B,),
        in_specs=[pl.BlockSpec((1, N, co), lambda b: (b, 0, 0)),
                  pl.BlockSpec((1, N, co), lambda b: (b, 0, 0)),
                  row1(), row1(), row1(), row1(), row1()],
        out_specs=[pl.BlockSpec((1, N, co), lambda b: (b, 0, 0)),
                   row1()],
        out_shape=[jax.ShapeDtypeStruct((B, N, co), f32),
                   jax.ShapeDtypeStruct((B, 1, co), f32)],
        compiler_params=pltpu.CompilerParams(
            dimension_semantics=("arbitrary",)),
    )(zmax, zmin, s1, s2, gam, bet, se)
    return y, o


def kernel(x, style, W1, S1w, S1b, F1a, F1b, W2, S2w, S2b, F2a, F2b,
           W3, S3w, S3b, F3a, F3b, W4, S4w, S4b, F4a, F4b):
    B = x.shape[0]
    gs = []
    for (Sw, Sb, Fa, Fb) in ((S1w, S1b, F1a, F1b), (S2w, S2b, F2a, F2b),
                             (S3w, S3b, F3a, F3b), (S4w, S4b, F4a, F4b)):
        co = Sw.shape[0] // 2
        st = style @ Sw.T + Sb
        gamma, beta = st[:, :co], st[:, co:]
        s = jax.nn.sigmoid(jnp.maximum(beta @ Fa.T, 0.0) @ Fb.T)
        gs.append((gamma.reshape(B, 1, co), beta.reshape(B, 1, co),
                   s.reshape(B, 1, co)))

    xt = jnp.transpose(x, (0, 2, 1))
    y1, o1 = _layer_call(xt, *gs[0], W1, 3, 64)
    y2, o2 = _layer_call(y1, *gs[1], W2, 64, 128)
    y3, o3 = _layer_call(y2, *gs[2], W3, 128, 256)
    _, o4 = _layer_call(y3, *gs[3], W4, 256, 512)
    return jnp.concatenate([o1[:, 0], o2[:, 0], o3[:, 0], o4[:, 0]], axis=1)
